# baseline (speedup 1.0000x reference)
import functools

import jax
import jax.numpy as jnp
from jax import lax
from jax.experimental import pallas as pl
from jax.experimental.pallas import tpu as pltpu
from jax.experimental.pallas import tpu_sc as plsc

N = 10000
E = 320000
D_IN = 128
D_OUT = 64

NC = 2
NS = 16
NW = NC * NS
L = 16

NPAD = 10240
ROWS_PER_TILE = NPAD // NS
K = 128
E_TOT = E
BLK_PAIR = 2 * (-(-E_TOT // (NS * K * 2)))
NBLK0 = 92
NBLK1 = BLK_PAIR - NBLK0
NBLK_MAX = max(NBLK0, NBLK1)
E_PAD = NS * BLK_PAIR * K
ACC_W = 80
MM_BLK = 256


def _mm_body(n_ref, wl_ref, wr_ref, wlin_ref, b_ref, xl_ref, xr_ref, lin_ref):
    n = n_ref[...]
    xl_ref[...] = jnp.dot(n, wl_ref[...], preferred_element_type=jnp.float32)
    xr_ref[...] = jnp.dot(n, wr_ref[...], preferred_element_type=jnp.float32)
    lin_ref[...] = (
        jnp.dot(n, wlin_ref[...], preferred_element_type=jnp.float32)
        + b_ref[0:1, :]
    )


def _node_transforms(node_p, Wl, Wr, Wlin, bias2):
    grid = (NPAD // MM_BLK,)
    w_spec = pl.BlockSpec((D_IN, D_OUT), lambda i: (0, 0))
    o_spec = pl.BlockSpec((MM_BLK, D_OUT), lambda i: (i, 0))
    return pl.pallas_call(
        _mm_body,
        grid=grid,
        in_specs=[
            pl.BlockSpec((MM_BLK, D_IN), lambda i: (i, 0)),
            w_spec, w_spec, w_spec,
            pl.BlockSpec((8, D_OUT), lambda i: (0, 0)),
        ],
        out_specs=[o_spec, o_spec, o_spec],
        out_shape=[jax.ShapeDtypeStruct((NPAD, D_OUT), jnp.float32)] * 3,
    )(node_p, Wl, Wr, Wlin, bias2)


def _sc_edge_body(xl_hbm, xr_hbm, src_hbm, dst_hbm, att_hbm, out_hbm,
                  att_v, idx_s, idx_d, rows_l0, rows_l1, rows_r0, rows_r1,
                  blk, accum, sl0, sl1, sr0, sr1):
    c = lax.axis_index("c")
    s = lax.axis_index("s")
    nhalf = jnp.where(c == 0, NBLK0 // 2, NBLK1 // 2)
    start_blk = jnp.where(c == 0, s * NBLK0, NS * NBLK0 + s * NBLK1)

    zeros16 = jnp.zeros((L,), jnp.float32)
    rows_l = (rows_l0, rows_l1)
    rows_r = (rows_r0, rows_r1)
    sl = (sl0, sl1)
    sr = (sr0, sr1)

    def _zrow(i, _):
        for cc in range(ACC_W // L):
            blk[i, pl.ds(cc * L, L)] = zeros16
        return _

    lax.fori_loop(0, K, _zrow, None)
    for j in range(ROWS_PER_TILE // K):
        pltpu.sync_copy(blk, accum.at[pl.ds(s * ROWS_PER_TILE + j * K, K)])
    plsc.subcore_barrier()

    pltpu.sync_copy(att_hbm, att_v)
    att_c = [att_v[pl.ds(cc * L, L)] for cc in range(D_OUT // L)]
    lanes = lax.iota(jnp.int32, L)

    pltpu.sync_copy(src_hbm.at[pl.ds(start_blk, NBLK_MAX)], idx_s)
    pltpu.sync_copy(dst_hbm.at[pl.ds(start_blk, NBLK_MAX)], idx_d)

    def _issue(g, b):
        pltpu.async_copy(xl_hbm.at[idx_s.at[g]], rows_l[b], sl[b])
        pltpu.async_copy(xr_hbm.at[idx_d.at[g]], rows_r[b], sr[b])

    _issue(0, 0)
    _issue(1, 1)

    def _iter(i, _):
        for b in range(2):
            g = 2 * i + b
            pltpu.make_async_copy(
                xl_hbm.at[idx_s.at[g]], rows_l[b], sl[b]).wait()
            pltpu.make_async_copy(
                xr_hbm.at[idx_d.at[g]], rows_r[b], sr[b]).wait()

            @plsc.parallel_loop(0, K, 1, unroll=4)
            def _edge(e):
                a = [rows_l[b][e, pl.ds(cc * L, L)]
                     for cc in range(D_OUT // L)]
                tot = zeros16
                for cc in range(D_OUT // L):
                    v = a[cc] + rows_r[b][e, pl.ds(cc * L, L)]
                    tot = tot + jnp.maximum(v, v * 0.2) * att_c[cc]
                p = jnp.exp(jnp.full((L,), jnp.sum(tot), jnp.float32))
                for cc in range(D_OUT // L):
                    blk[e, pl.ds(cc * L, L)] = a[cc] * p
                blk[e, pl.ds(D_OUT, L)] = p

            @pl.when(i < nhalf - 1)
            def _():
                _issue(g + 2, b)

            pltpu.sync_copy(blk, accum.at[idx_d.at[g]], add=True)
        return _

    lax.fori_loop(0, nhalf, _iter, None)
    plsc.subcore_barrier()
    pltpu.sync_copy(
        accum.at[pl.ds(s * ROWS_PER_TILE, ROWS_PER_TILE)],
        out_hbm.at[c, pl.ds(s * ROWS_PER_TILE, ROWS_PER_TILE)],
    )


_sc_edge = functools.partial(
    pl.kernel,
    out_type=jax.ShapeDtypeStruct((NC, NPAD, ACC_W), jnp.float32),
    mesh=plsc.VectorSubcoreMesh(core_axis_name="c", subcore_axis_name="s"),
    compiler_params=pltpu.CompilerParams(
        needs_layout_passes=False, use_tc_tiling_on_sc=False),
    scratch_types=[
        pltpu.VMEM((D_OUT,), jnp.float32),
        pltpu.VMEM((NBLK_MAX, K), jnp.int32),
        pltpu.VMEM((NBLK_MAX, K), jnp.int32),
        pltpu.VMEM((K, D_OUT), jnp.float32),
        pltpu.VMEM((K, D_OUT), jnp.float32),
        pltpu.VMEM((K, D_OUT), jnp.float32),
        pltpu.VMEM((K, D_OUT), jnp.float32),
        pltpu.VMEM((K, ACC_W), jnp.float32),
        pltpu.VMEM_SHARED((NPAD, ACC_W), jnp.float32),
        pltpu.SemaphoreType.DMA,
        pltpu.SemaphoreType.DMA,
        pltpu.SemaphoreType.DMA,
        pltpu.SemaphoreType.DMA,
    ],
)(_sc_edge_body)


def _combine_body(p0_ref, p1_ref, lin_ref, xl_ref, xr_ref, att_ref, out_ref):
    p0 = p0_ref[0]
    p1 = p1_ref[0]
    num = p0[:, :D_OUT] + p1[:, :D_OUT]
    den = p0[:, D_OUT:D_OUT + 1] + p1[:, D_OUT:D_OUT + 1]
    xlv = xl_ref[...]
    v = xlv + xr_ref[...]
    e = jnp.maximum(v, 0.2 * v) * att_ref[0:1, :]
    p = jnp.exp(jnp.sum(e, axis=1, keepdims=True))
    num = num + p * xlv
    den = den + p
    out = num / jnp.maximum(den, 1e-30) + lin_ref[...]
    out_ref[...] = jnp.maximum(out, 0.0)


def _combine(acc, lin, xl, xr, attb):
    blk = 1000
    grid = (N // blk,)
    nspec = pl.BlockSpec((blk, D_OUT), lambda i: (i, 0))
    return pl.pallas_call(
        _combine_body,
        grid=grid,
        in_specs=[
            pl.BlockSpec((1, blk, ACC_W), lambda i: (0, i, 0)),
            pl.BlockSpec((1, blk, ACC_W), lambda i: (1, i, 0)),
            nspec, nspec, nspec,
            pl.BlockSpec((8, D_OUT), lambda i: (0, 0)),
        ],
        out_specs=nspec,
        out_shape=jax.ShapeDtypeStruct((N, D_OUT), jnp.float32),
    )(acc, acc, lin, xl, xr, attb)


def kernel(node, edge_index, Wl, Wr, att, bias, Wlin, blin):
    node_p = jnp.pad(node, ((0, NPAD - N), (0, 0)))
    bias2 = jnp.broadcast_to((bias + blin).reshape(1, D_OUT), (8, D_OUT))
    xl, xr, lin = _node_transforms(node_p, Wl, Wr, Wlin, bias2)

    pad_n = E_PAD - E_TOT
    src = jnp.pad(edge_index[0], (0, pad_n)).reshape(E_PAD // K, K)
    dst = jnp.pad(edge_index[1], (0, pad_n),
                  constant_values=N).reshape(E_PAD // K, K)

    acc = _sc_edge(xl, xr, src, dst, att)
    attb = jnp.broadcast_to(att.reshape(1, D_OUT), (8, D_OUT))
    return _combine(acc, lin, xl[:N], xr[:N], attb)


# speedup vs baseline: 1.1364x; 1.1364x over previous
import functools

import jax
import jax.numpy as jnp
from jax import lax
from jax.experimental import pallas as pl
from jax.experimental.pallas import tpu as pltpu
from jax.experimental.pallas import tpu_sc as plsc

N = 10000
E = 320000
D_IN = 128
D_OUT = 64

NC = 2
NS = 16
NW = NC * NS
L = 16

NPAD = 10240
ROWS_PER_TILE = NPAD // NS
K = 128
E_TOT = E + N
BLK_PAIR = 2 * (-(-E_TOT // (NS * K * 2)))
NBLK0 = 94
NBLK1 = BLK_PAIR - NBLK0
NBLK_MAX = max(NBLK0, NBLK1)
E_PAD = NS * BLK_PAIR * K
ACC_W = 80
MM_BLK = 256


def _mm_body(n_ref, wl_ref, wr_ref, wlin_ref, b_ref, xl_ref, xr_ref, lin_ref):
    n = n_ref[...]
    xl_ref[...] = jnp.dot(n, wl_ref[...], preferred_element_type=jnp.float32)
    xr_ref[...] = jnp.dot(n, wr_ref[...], preferred_element_type=jnp.float32)
    lin_ref[...] = (
        jnp.dot(n, wlin_ref[...], preferred_element_type=jnp.float32)
        + b_ref[0:1, :]
    )


def _node_transforms(node_p, Wl, Wr, Wlin, bias2):
    grid = (NPAD // MM_BLK,)
    w_spec = pl.BlockSpec((D_IN, D_OUT), lambda i: (0, 0))
    o_spec = pl.BlockSpec((MM_BLK, D_OUT), lambda i: (i, 0))
    return pl.pallas_call(
        _mm_body,
        grid=grid,
        in_specs=[
            pl.BlockSpec((MM_BLK, D_IN), lambda i: (i, 0)),
            w_spec, w_spec, w_spec,
            pl.BlockSpec((8, D_OUT), lambda i: (0, 0)),
        ],
        out_specs=[o_spec, o_spec, o_spec],
        out_shape=[jax.ShapeDtypeStruct((NPAD, D_OUT), jnp.float32)] * 3,
    )(node_p, Wl, Wr, Wlin, bias2)


def _sc_edge_body(xl_hbm, xr_hbm, src_hbm, dst_hbm, att_hbm, out_hbm,
                  att_v, idx_s, idx_d, rows_l0, rows_l1, rows_r0, rows_r1,
                  blk, accum, sl0, sl1, sr0, sr1):
    c = lax.axis_index("c")
    s = lax.axis_index("s")
    nhalf = jnp.where(c == 0, NBLK0 // 2, NBLK1 // 2)
    start_blk = jnp.where(c == 0, s * NBLK0, NS * NBLK0 + s * NBLK1)

    zeros16 = jnp.zeros((L,), jnp.float32)
    rows_l = (rows_l0, rows_l1)
    rows_r = (rows_r0, rows_r1)
    sl = (sl0, sl1)
    sr = (sr0, sr1)

    def _zrow(i, _):
        for cc in range(ACC_W // L):
            blk[i, pl.ds(cc * L, L)] = zeros16
        return _

    lax.fori_loop(0, K, _zrow, None)
    for j in range(ROWS_PER_TILE // K):
        pltpu.sync_copy(blk, accum.at[pl.ds(s * ROWS_PER_TILE + j * K, K)])
    plsc.subcore_barrier()

    pltpu.sync_copy(att_hbm, att_v)
    att_c = [att_v[pl.ds(cc * L, L)] for cc in range(D_OUT // L)]
    lanes = lax.iota(jnp.int32, L)

    pltpu.sync_copy(src_hbm.at[pl.ds(start_blk, NBLK_MAX)], idx_s)
    pltpu.sync_copy(dst_hbm.at[pl.ds(start_blk, NBLK_MAX)], idx_d)

    def _issue(g, b):
        pltpu.async_copy(xl_hbm.at[idx_s.at[g]], rows_l[b], sl[b])
        pltpu.async_copy(xr_hbm.at[idx_d.at[g]], rows_r[b], sr[b])

    _issue(0, 0)
    _issue(1, 1)

    def _iter(i, _):
        for b in range(2):
            g = 2 * i + b
            pltpu.make_async_copy(
                xl_hbm.at[idx_s.at[g]], rows_l[b], sl[b]).wait()
            pltpu.make_async_copy(
                xr_hbm.at[idx_d.at[g]], rows_r[b], sr[b]).wait()

            @plsc.parallel_loop(0, K, 1, unroll=4)
            def _edge(e):
                a = [rows_l[b][e, pl.ds(cc * L, L)]
                     for cc in range(D_OUT // L)]
                tot = zeros16
                for cc in range(D_OUT // L):
                    v = a[cc] + rows_r[b][e, pl.ds(cc * L, L)]
                    tot = tot + jnp.maximum(v, v * 0.2) * att_c[cc]
                p = jnp.exp(jnp.full((L,), jnp.sum(tot), jnp.float32))
                for cc in range(D_OUT // L):
                    blk[e, pl.ds(cc * L, L)] = a[cc] * p
                blk[e, pl.ds(D_OUT, L)] = p

            @pl.when(i < nhalf - 1)
            def _():
                _issue(g + 2, b)

            pltpu.sync_copy(blk, accum.at[idx_d.at[g]], add=True)
        return _

    lax.fori_loop(0, nhalf, _iter, None)
    plsc.subcore_barrier()
    pltpu.sync_copy(
        accum.at[pl.ds(s * ROWS_PER_TILE, ROWS_PER_TILE)],
        out_hbm.at[c, pl.ds(s * ROWS_PER_TILE, ROWS_PER_TILE)],
    )


_sc_edge = functools.partial(
    pl.kernel,
    out_type=jax.ShapeDtypeStruct((NC, NPAD, ACC_W), jnp.float32),
    mesh=plsc.VectorSubcoreMesh(core_axis_name="c", subcore_axis_name="s"),
    compiler_params=pltpu.CompilerParams(
        needs_layout_passes=False, use_tc_tiling_on_sc=False),
    scratch_types=[
        pltpu.VMEM((D_OUT,), jnp.float32),
        pltpu.VMEM((NBLK_MAX, K), jnp.int32),
        pltpu.VMEM((NBLK_MAX, K), jnp.int32),
        pltpu.VMEM((K, D_OUT), jnp.float32),
        pltpu.VMEM((K, D_OUT), jnp.float32),
        pltpu.VMEM((K, D_OUT), jnp.float32),
        pltpu.VMEM((K, D_OUT), jnp.float32),
        pltpu.VMEM((K, ACC_W), jnp.float32),
        pltpu.VMEM_SHARED((NPAD, ACC_W), jnp.float32),
        pltpu.SemaphoreType.DMA,
        pltpu.SemaphoreType.DMA,
        pltpu.SemaphoreType.DMA,
        pltpu.SemaphoreType.DMA,
    ],
)(_sc_edge_body)


def _combine_body(p0_ref, p1_ref, lin_ref, out_ref):
    p0 = p0_ref[0]
    p1 = p1_ref[0]
    num = p0[:, :D_OUT] + p1[:, :D_OUT]
    den = p0[:, D_OUT:D_OUT + 1] + p1[:, D_OUT:D_OUT + 1]
    out = num / jnp.maximum(den, 1e-30) + lin_ref[...]
    out_ref[...] = jnp.maximum(out, 0.0)


def _combine(acc, lin):
    blk = 1000
    grid = (N // blk,)
    return pl.pallas_call(
        _combine_body,
        grid=grid,
        in_specs=[
            pl.BlockSpec((1, blk, ACC_W), lambda i: (0, i, 0)),
            pl.BlockSpec((1, blk, ACC_W), lambda i: (1, i, 0)),
            pl.BlockSpec((blk, D_OUT), lambda i: (i, 0)),
        ],
        out_specs=pl.BlockSpec((blk, D_OUT), lambda i: (i, 0)),
        out_shape=jax.ShapeDtypeStruct((N, D_OUT), jnp.float32),
    )(acc, acc, lin)


def kernel(node, edge_index, Wl, Wr, att, bias, Wlin, blin):
    node_p = jnp.pad(node, ((0, NPAD - N), (0, 0)))
    bias2 = jnp.broadcast_to((bias + blin).reshape(1, D_OUT), (8, D_OUT))
    xl, xr, lin = _node_transforms(node_p, Wl, Wr, Wlin, bias2)

    loops = jnp.arange(N, dtype=jnp.int32)
    pad_n = E_PAD - E_TOT
    src = jnp.concatenate(
        [edge_index[0], loops, jnp.zeros((pad_n,), jnp.int32)])
    dst = jnp.concatenate(
        [edge_index[1], loops, jnp.full((pad_n,), N, jnp.int32)])
    src = src.reshape(E_PAD // K, K)
    dst = dst.reshape(E_PAD // K, K)

    acc = _sc_edge(xl, xr, src, dst, att)
    return _combine(acc, lin)


# speedup vs baseline: 1.1366x; 1.0002x over previous
import functools

import jax
import jax.numpy as jnp
from jax import lax
from jax.experimental import pallas as pl
from jax.experimental.pallas import tpu as pltpu
from jax.experimental.pallas import tpu_sc as plsc

N = 10000
E = 320000
D_IN = 128
D_OUT = 64

NC = 2
NS = 16
NW = NC * NS
L = 16

NPAD = 10240
ROWS_PER_TILE = NPAD // NS
K = 128
E_TOT = E + N
BLK_PAIR = 2 * (-(-E_TOT // (NS * K * 2)))
NBLK0 = 94
NBLK1 = BLK_PAIR - NBLK0
NBLK_MAX = max(NBLK0, NBLK1)
E_PAD = NS * BLK_PAIR * K
ACC_W = 80
MM_BLK = 256


def _mm_body(n_ref, wl_ref, wr_ref, wlin_ref, b_ref, xl_ref, xr_ref, lin_ref):
    n = n_ref[...]
    xl_ref[...] = jnp.dot(n, wl_ref[...], preferred_element_type=jnp.float32)
    xr_ref[...] = jnp.dot(n, wr_ref[...], preferred_element_type=jnp.float32)
    lin_ref[...] = (
        jnp.dot(n, wlin_ref[...], preferred_element_type=jnp.float32)
        + b_ref[0:1, :]
    )


def _node_transforms(node_p, Wl, Wr, Wlin, bias2):
    grid = (NPAD // MM_BLK,)
    w_spec = pl.BlockSpec((D_IN, D_OUT), lambda i: (0, 0))
    o_spec = pl.BlockSpec((MM_BLK, D_OUT), lambda i: (i, 0))
    return pl.pallas_call(
        _mm_body,
        grid=grid,
        in_specs=[
            pl.BlockSpec((MM_BLK, D_IN), lambda i: (i, 0)),
            w_spec, w_spec, w_spec,
            pl.BlockSpec((8, D_OUT), lambda i: (0, 0)),
        ],
        out_specs=[o_spec, o_spec, o_spec],
        out_shape=[jax.ShapeDtypeStruct((NPAD, D_OUT), jnp.float32)] * 3,
    )(node_p, Wl, Wr, Wlin, bias2)


def _sc_edge_body(xl_hbm, xr_hbm, src_hbm, dst_hbm, att_hbm, out_hbm,
                  att_v, idx_s, idx_d, rows_l0, rows_l1, rows_r0, rows_r1,
                  blk, accum, sl0, sl1, sr0, sr1):
    c = lax.axis_index("c")
    s = lax.axis_index("s")
    nhalf = jnp.where(c == 0, NBLK0 // 2, NBLK1 // 2)
    start_blk = jnp.where(c == 0, s * NBLK0, NS * NBLK0 + s * NBLK1)

    zeros16 = jnp.zeros((L,), jnp.float32)
    rows_l = (rows_l0, rows_l1)
    rows_r = (rows_r0, rows_r1)
    sl = (sl0, sl1)
    sr = (sr0, sr1)

    def _zrow(i, _):
        for cc in range(ACC_W // L):
            blk[i, pl.ds(cc * L, L)] = zeros16
        return _

    lax.fori_loop(0, K, _zrow, None)
    for j in range(ROWS_PER_TILE // K):
        pltpu.sync_copy(blk, accum.at[pl.ds(s * ROWS_PER_TILE + j * K, K)])
    plsc.subcore_barrier()

    pltpu.sync_copy(att_hbm, att_v)
    att_c = [att_v[pl.ds(cc * L, L)] for cc in range(D_OUT // L)]

    @pl.when(c == 0)
    def _():
        pltpu.sync_copy(src_hbm.at[pl.ds(start_blk, NBLK0)],
                        idx_s.at[pl.ds(0, NBLK0)])
        pltpu.sync_copy(dst_hbm.at[pl.ds(start_blk, NBLK0)],
                        idx_d.at[pl.ds(0, NBLK0)])

    @pl.when(c == 1)
    def _():
        pltpu.sync_copy(src_hbm.at[pl.ds(start_blk, NBLK1)],
                        idx_s.at[pl.ds(0, NBLK1)])
        pltpu.sync_copy(dst_hbm.at[pl.ds(start_blk, NBLK1)],
                        idx_d.at[pl.ds(0, NBLK1)])

    def _issue(g, b):
        pltpu.async_copy(xl_hbm.at[idx_s.at[g]], rows_l[b], sl[b])
        pltpu.async_copy(xr_hbm.at[idx_d.at[g]], rows_r[b], sr[b])

    _issue(0, 0)
    _issue(1, 1)

    def _iter(i, _):
        for b in range(2):
            g = 2 * i + b
            pltpu.make_async_copy(
                xl_hbm.at[idx_s.at[g]], rows_l[b], sl[b]).wait()
            pltpu.make_async_copy(
                xr_hbm.at[idx_d.at[g]], rows_r[b], sr[b]).wait()

            @plsc.parallel_loop(0, K, 1, unroll=4)
            def _edge(e):
                a = [rows_l[b][e, pl.ds(cc * L, L)]
                     for cc in range(D_OUT // L)]
                tot = zeros16
                for cc in range(D_OUT // L):
                    v = a[cc] + rows_r[b][e, pl.ds(cc * L, L)]
                    tot = tot + jnp.maximum(v, v * 0.2) * att_c[cc]
                p = jnp.exp(jnp.full((L,), jnp.sum(tot), jnp.float32))
                for cc in range(D_OUT // L):
                    blk[e, pl.ds(cc * L, L)] = a[cc] * p
                blk[e, pl.ds(D_OUT, L)] = p

            @pl.when(i < nhalf - 1)
            def _():
                _issue(g + 2, b)

            pltpu.sync_copy(blk, accum.at[idx_d.at[g]], add=True)
        return _

    lax.fori_loop(0, nhalf, _iter, None)
    plsc.subcore_barrier()
    pltpu.sync_copy(
        accum.at[pl.ds(s * ROWS_PER_TILE, ROWS_PER_TILE)],
        out_hbm.at[c, pl.ds(s * ROWS_PER_TILE, ROWS_PER_TILE)],
    )


_sc_edge = functools.partial(
    pl.kernel,
    out_type=jax.ShapeDtypeStruct((NC, NPAD, ACC_W), jnp.float32),
    mesh=plsc.VectorSubcoreMesh(core_axis_name="c", subcore_axis_name="s"),
    compiler_params=pltpu.CompilerParams(
        needs_layout_passes=False, use_tc_tiling_on_sc=False),
    scratch_types=[
        pltpu.VMEM((D_OUT,), jnp.float32),
        pltpu.VMEM((NBLK_MAX, K), jnp.int32),
        pltpu.VMEM((NBLK_MAX, K), jnp.int32),
        pltpu.VMEM((K, D_OUT), jnp.float32),
        pltpu.VMEM((K, D_OUT), jnp.float32),
        pltpu.VMEM((K, D_OUT), jnp.float32),
        pltpu.VMEM((K, D_OUT), jnp.float32),
        pltpu.VMEM((K, ACC_W), jnp.float32),
        pltpu.VMEM_SHARED((NPAD, ACC_W), jnp.float32),
        pltpu.SemaphoreType.DMA,
        pltpu.SemaphoreType.DMA,
        pltpu.SemaphoreType.DMA,
        pltpu.SemaphoreType.DMA,
    ],
)(_sc_edge_body)


def _combine_body(p0_ref, p1_ref, lin_ref, out_ref):
    p0 = p0_ref[0]
    p1 = p1_ref[0]
    num = p0[:, :D_OUT] + p1[:, :D_OUT]
    den = p0[:, D_OUT:D_OUT + 1] + p1[:, D_OUT:D_OUT + 1]
    out = num / jnp.maximum(den, 1e-30) + lin_ref[...]
    out_ref[...] = jnp.maximum(out, 0.0)


def _combine(acc, lin):
    blk = 1000
    grid = (N // blk,)
    return pl.pallas_call(
        _combine_body,
        grid=grid,
        in_specs=[
            pl.BlockSpec((1, blk, ACC_W), lambda i: (0, i, 0)),
            pl.BlockSpec((1, blk, ACC_W), lambda i: (1, i, 0)),
            pl.BlockSpec((blk, D_OUT), lambda i: (i, 0)),
        ],
        out_specs=pl.BlockSpec((blk, D_OUT), lambda i: (i, 0)),
        out_shape=jax.ShapeDtypeStruct((N, D_OUT), jnp.float32),
    )(acc, acc, lin)


def kernel(node, edge_index, Wl, Wr, att, bias, Wlin, blin):
    node_p = jnp.pad(node, ((0, NPAD - N), (0, 0)))
    bias2 = jnp.broadcast_to((bias + blin).reshape(1, D_OUT), (8, D_OUT))
    xl, xr, lin = _node_transforms(node_p, Wl, Wr, Wlin, bias2)

    loops = jnp.arange(N, dtype=jnp.int32)
    pad_n = E_PAD - E_TOT
    src = jnp.concatenate(
        [edge_index[0], loops, jnp.zeros((pad_n,), jnp.int32)])
    dst = jnp.concatenate(
        [edge_index[1], loops, jnp.full((pad_n,), N, jnp.int32)])
    src = src.reshape(E_PAD // K, K)
    dst = dst.reshape(E_PAD // K, K)

    acc = _sc_edge(xl, xr, src, dst, att)
    return _combine(acc, lin)
